# baseline (speedup 1.0000x reference)
import jax
import jax.numpy as jnp
from jax.experimental import pallas as pl
from jax.experimental.pallas import tpu as pltpu

B, K, V = 32, 8, 100000
PAD_TOKEN_ID = 0
KP = K + 1
LT = 128
SEGS = 4
CW = SEGS * V
RVIEW = B * KP // SEGS
RB = 8
UNROLL = 4
NINT = 780 // UNROLL
IBIG = 2**31 - 1


def _scan_kernel(tp_ref, ti_ref, did_ref, dp_ref, tid_out, tps_out):
    lane2d = jax.lax.broadcasted_iota(jnp.int32, (RB, LT), 1)

    for j in range(SEGS):
        did = did_ref[:, j * LT:j * LT + 1]
        dp = dp_ref[:, j * LT:j * LT + 1]
        off = j * V
        t0 = off // LT
        o0 = off % LT
        t_end = (off + V) // LT
        o_end = (off + V) % LT

        def tile(t, valid=None):
            tp = tp_ref[:, pl.ds(t * LT, LT)]
            ti = ti_ref[:, pl.ds(t * LT, LT)]
            eq = ti == did
            if valid is not None:
                eq = eq & valid
                tp = jnp.where(valid, tp, 0.0)
            c = jnp.maximum(tp - jnp.where(eq, dp, 0.0), 0.0)
            msel = jnp.where(eq, tp, 0.0)
            return ti, c, msel

        def update(state, t, ti, c, msel):
            s_max, s_vt, s_ti, s_sum, s_tps = state
            s_sum = s_sum + c
            s_tps = s_tps + msel
            upd = c > s_max
            s_max = jnp.where(upd, c, s_max)
            s_vt = jnp.where(upd, t, s_vt)
            s_ti = jnp.where(upd, ti, s_ti)
            return (s_max, s_vt, s_ti, s_sum, s_tps)

        v0 = None if o0 == 0 else (lane2d >= o0)
        ti0, c0, msel0 = tile(t0, v0)
        state = (c0, jnp.full((RB, LT), t0, jnp.int32), ti0, c0, msel0)

        def body(i, state):
            for u in range(UNROLL):
                t = t0 + 1 + i * UNROLL + u
                state = update(state, t, *tile(t))
            return state

        state = jax.lax.fori_loop(0, NINT, body, state)
        for t in range(t0 + 1 + NINT * UNROLL, t_end):
            state = update(state, t, *tile(t))

        if o_end != 0:
            state = update(state, t_end, *tile(t_end, lane2d < o_end))

        s_max, s_vt, s_ti, s_sum, s_tps = state

        m = jnp.max(s_max, axis=1, keepdims=True)
        colg = s_vt * LT + lane2d - off
        idx = jnp.min(jnp.where(s_max == m, colg, IBIG), axis=1,
                      keepdims=True)
        tiv = jnp.sum(jnp.where(colg == idx, s_ti, 0), axis=1,
                      keepdims=True)
        ssum = jnp.sum(s_sum, axis=1, keepdims=True)
        tps = jnp.sum(s_tps, axis=1, keepdims=True)
        ti0c = jnp.sum(jnp.where(colg == 0, s_ti, 0), axis=1,
                       keepdims=True)
        tid = jnp.where(ssum < 1e-30, ti0c, tiv)
        tid_out[:, pl.ds(j * LT, LT)] = jnp.broadcast_to(tid, (RB, LT))
        tps_out[:, pl.ds(j * LT, LT)] = jnp.broadcast_to(tps, (RB, LT))


def _tail_kernel(tid_ref, tps_ref, did_ref, dp_ref, rnd_ref,
                 tok_out, idx_out):
    tid = tid_ref[...]
    tps = tps_ref[...]
    did = did_ref[...]
    dp = dp_ref[...]
    rnd = rnd_ref[...]

    ratio = jnp.minimum(tps / dp, 1.0)
    acc = (rnd < ratio).astype(jnp.float32)

    r_i = jax.lax.broadcasted_iota(jnp.int32, (16, 16), 0)
    c_i = jax.lax.broadcasted_iota(jnp.int32, (16, 16), 1)
    lt = (r_i <= c_i).astype(jnp.float32)
    cs = jnp.dot(acc, lt, preferred_element_type=jnp.float32)

    lane = jax.lax.broadcasted_iota(jnp.int32, (32, 16), 1)
    positions = (lane + 1).astype(jnp.float32)
    mask = cs == positions
    index = jnp.sum(mask.astype(jnp.int32), axis=1, keepdims=True)

    tokens = jnp.where(mask, did, tid)
    keep = index >= lane
    tokens = jnp.where(keep, tokens, PAD_TOKEN_ID)

    tok_out[...] = tokens
    idx_out[...] = jnp.broadcast_to(index, (32, 16))


@jax.jit
def kernel(draft_ids, draft_probs, target_probs, target_indices):
    tp_v = target_probs.reshape(RVIEW, CW)
    ti_v = target_indices.reshape(RVIEW, CW)

    did_rows = jnp.concatenate(
        [draft_ids, jnp.full((B, 1), -1, jnp.int32)], axis=1)
    dp_rows = jnp.concatenate(
        [draft_probs, jnp.zeros((B, 1), jnp.float32)], axis=1)
    did_b = jnp.broadcast_to(
        did_rows.reshape(RVIEW, SEGS)[:, :, None],
        (RVIEW, SEGS, LT)).reshape(RVIEW, SEGS * LT)
    dp_b = jnp.broadcast_to(
        dp_rows.reshape(RVIEW, SEGS)[:, :, None],
        (RVIEW, SEGS, LT)).reshape(RVIEW, SEGS * LT)

    tid_full, tps_full = pl.pallas_call(
        _scan_kernel,
        grid=(RVIEW // RB,),
        in_specs=[
            pl.BlockSpec((RB, CW), lambda r: (r, 0)),
            pl.BlockSpec((RB, CW), lambda r: (r, 0)),
            pl.BlockSpec((RB, SEGS * LT), lambda r: (r, 0)),
            pl.BlockSpec((RB, SEGS * LT), lambda r: (r, 0)),
        ],
        out_specs=[
            pl.BlockSpec((RB, SEGS * LT), lambda r: (r, 0)),
            pl.BlockSpec((RB, SEGS * LT), lambda r: (r, 0)),
        ],
        out_shape=[
            jax.ShapeDtypeStruct((RVIEW, SEGS * LT), jnp.int32),
            jax.ShapeDtypeStruct((RVIEW, SEGS * LT), jnp.float32),
        ],
        compiler_params=pltpu.CompilerParams(
            dimension_semantics=("arbitrary",)),
    )(tp_v, ti_v, did_b, dp_b)

    target_ids = tid_full.reshape(RVIEW, SEGS, LT)[:, :, 0].reshape(B, KP)
    tp_sel = tps_full.reshape(RVIEW, SEGS, LT)[:, :, 0].reshape(B, KP)[:, :K]

    tid_p = jnp.pad(target_ids, ((0, 0), (0, 16 - KP)))
    tps_p = jnp.pad(tp_sel, ((0, 0), (0, 16 - K)))
    did_p = jnp.pad(draft_ids, ((0, 0), (0, 16 - K)))
    dp_p = jnp.pad(draft_probs, ((0, 0), (0, 16 - K)), constant_values=1.0)
    rnd = jax.random.uniform(jax.random.key(42), (B, K), dtype=jnp.float32)
    rnd_p = jnp.pad(rnd, ((0, 0), (0, 16 - K)), constant_values=2.0)

    tokens16, idx16 = pl.pallas_call(
        _tail_kernel,
        out_shape=[
            jax.ShapeDtypeStruct((B, 16), jnp.int32),
            jax.ShapeDtypeStruct((B, 16), jnp.int32),
        ],
    )(tid_p, tps_p, did_p, dp_p, rnd_p)

    tokens = tokens16[:, :KP]
    index = idx16[:, :1]
    return (tokens, index)


# speedup vs baseline: 26.7965x; 26.7965x over previous
import jax
import jax.numpy as jnp
from jax.experimental import pallas as pl
from jax.experimental.pallas import tpu as pltpu

B, K, V = 32, 8, 100000
PAD_TOKEN_ID = 0
KP = K + 1
LT = 128
FULL = V // LT
REM = V - FULL * LT
UNROLL = 4
NLOOP = (FULL - 1) // UNROLL
IBIG = 2**31 - 1


def _scan_kernel(tp_ref, ti_ref, did_ref, dp_ref, tid_out, tps_out):
    lane2d = jax.lax.broadcasted_iota(jnp.int32, (8, LT), 1)

    for q in range(B // 8):
        r0 = 8 * q
        did = did_ref[0, r0:r0 + 8, :1]
        dp = dp_ref[0, r0:r0 + 8, :1]

        def tile(t):
            tp = tp_ref[0, r0:r0 + 8, pl.ds(t * LT, LT)]
            ti = ti_ref[0, r0:r0 + 8, pl.ds(t * LT, LT)]
            eq = ti == did
            c = jnp.maximum(tp - jnp.where(eq, dp, 0.0), 0.0)
            msel = jnp.where(eq, tp, 0.0)
            return ti, c, msel

        def update(state, t, ti, c, msel):
            s_max, s_vt, s_ti, s_sum, s_tps = state
            s_sum = s_sum + c
            s_tps = s_tps + msel
            upd = c > s_max
            s_max = jnp.where(upd, c, s_max)
            s_vt = jnp.where(upd, t, s_vt)
            s_ti = jnp.where(upd, ti, s_ti)
            return (s_max, s_vt, s_ti, s_sum, s_tps)

        ti0, c0, msel0 = tile(0)
        state = (c0, jnp.zeros((8, LT), jnp.int32), ti0, c0, msel0)

        def body(i, state):
            for u in range(UNROLL):
                t = 1 + i * UNROLL + u
                state = update(state, t, *tile(t))
            return state

        state = jax.lax.fori_loop(0, NLOOP, body, state)

        tp_r = tp_ref[0, r0:r0 + 8, pl.ds(FULL * LT, REM)]
        ti_r = ti_ref[0, r0:r0 + 8, pl.ds(FULL * LT, REM)]
        eq_r = ti_r == did
        c_r = jnp.maximum(tp_r - jnp.where(eq_r, dp, 0.0), 0.0)
        msel_r = jnp.where(eq_r, tp_r, 0.0)
        zf = jnp.zeros((8, LT - REM), jnp.float32)
        zi = jnp.zeros((8, LT - REM), jnp.int32)
        c = jnp.concatenate([c_r, zf], axis=1)
        msel = jnp.concatenate([msel_r, zf], axis=1)
        ti = jnp.concatenate([ti_r, zi], axis=1)
        s_max, s_vt, s_ti, s_sum, s_tps = update(state, FULL, ti, c, msel)

        m = jnp.max(s_max, axis=1, keepdims=True)
        colg = s_vt * LT + lane2d
        idx = jnp.min(jnp.where(s_max == m, colg, IBIG), axis=1,
                      keepdims=True)
        tiv = jnp.sum(jnp.where(colg == idx, s_ti, 0), axis=1,
                      keepdims=True)
        ssum = jnp.sum(s_sum, axis=1, keepdims=True)
        tps = jnp.sum(s_tps, axis=1, keepdims=True)
        ti0c = jnp.sum(jnp.where(colg == 0, s_ti, 0), axis=1,
                       keepdims=True)
        tid = jnp.where(ssum < 1e-30, ti0c, tiv)
        tid_out[0, r0:r0 + 8, :] = jnp.broadcast_to(tid, (8, LT))
        tps_out[0, r0:r0 + 8, :] = jnp.broadcast_to(tps, (8, LT))


def _tail_kernel(tid_ref, tps_ref, did_ref, dp_ref, rnd_ref,
                 tok_out, idx_out):
    tid = tid_ref[...]
    tps = tps_ref[...]
    did = did_ref[...]
    dp = dp_ref[...]
    rnd = rnd_ref[...]

    ratio = jnp.minimum(tps / dp, 1.0)
    acc = (rnd < ratio).astype(jnp.float32)

    r_i = jax.lax.broadcasted_iota(jnp.int32, (16, 16), 0)
    c_i = jax.lax.broadcasted_iota(jnp.int32, (16, 16), 1)
    lt = (r_i <= c_i).astype(jnp.float32)
    cs = jnp.dot(acc, lt, preferred_element_type=jnp.float32)

    lane = jax.lax.broadcasted_iota(jnp.int32, (32, 16), 1)
    positions = (lane + 1).astype(jnp.float32)
    mask = cs == positions
    index = jnp.sum(mask.astype(jnp.int32), axis=1, keepdims=True)

    tokens = jnp.where(mask, did, tid)
    keep = index >= lane
    tokens = jnp.where(keep, tokens, PAD_TOKEN_ID)

    tok_out[...] = tokens
    idx_out[...] = jnp.broadcast_to(index, (32, 16))


@jax.jit
def kernel(draft_ids, draft_probs, target_probs, target_indices):
    tp_t = jnp.transpose(target_probs, (1, 0, 2))
    ti_t = jnp.transpose(target_indices, (1, 0, 2))

    didT = jnp.concatenate(
        [draft_ids, jnp.full((B, 1), -1, jnp.int32)], axis=1).T
    dpT = jnp.concatenate(
        [draft_probs, jnp.zeros((B, 1), jnp.float32)], axis=1).T
    did_b = jnp.broadcast_to(didT[:, :, None], (KP, B, 128))
    dp_b = jnp.broadcast_to(dpT[:, :, None], (KP, B, 128))

    tid_full, tps_full = pl.pallas_call(
        _scan_kernel,
        grid=(KP,),
        in_specs=[
            pl.BlockSpec((1, B, V), lambda k: (k, 0, 0)),
            pl.BlockSpec((1, B, V), lambda k: (k, 0, 0)),
            pl.BlockSpec((1, B, 128), lambda k: (k, 0, 0)),
            pl.BlockSpec((1, B, 128), lambda k: (k, 0, 0)),
        ],
        out_specs=[
            pl.BlockSpec((1, B, 128), lambda k: (k, 0, 0)),
            pl.BlockSpec((1, B, 128), lambda k: (k, 0, 0)),
        ],
        out_shape=[
            jax.ShapeDtypeStruct((KP, B, 128), jnp.int32),
            jax.ShapeDtypeStruct((KP, B, 128), jnp.float32),
        ],
        compiler_params=pltpu.CompilerParams(
            dimension_semantics=("arbitrary",)),
    )(tp_t, ti_t, did_b, dp_b)

    target_ids = tid_full[:, :, 0].T
    tp_sel = tps_full[:, :, 0].T[:, :K]

    tid_p = jnp.pad(target_ids, ((0, 0), (0, 16 - KP)))
    tps_p = jnp.pad(tp_sel, ((0, 0), (0, 16 - K)))
    did_p = jnp.pad(draft_ids, ((0, 0), (0, 16 - K)))
    dp_p = jnp.pad(draft_probs, ((0, 0), (0, 16 - K)), constant_values=1.0)
    rnd = jax.random.uniform(jax.random.key(42), (B, K), dtype=jnp.float32)
    rnd_p = jnp.pad(rnd, ((0, 0), (0, 16 - K)), constant_values=2.0)

    tokens16, idx16 = pl.pallas_call(
        _tail_kernel,
        out_shape=[
            jax.ShapeDtypeStruct((B, 16), jnp.int32),
            jax.ShapeDtypeStruct((B, 16), jnp.int32),
        ],
    )(tid_p, tps_p, did_p, dp_p, rnd_p)

    tokens = tokens16[:, :KP]
    index = idx16[:, :1]
    return (tokens, index)


# speedup vs baseline: 32.5632x; 1.2152x over previous
import jax
import jax.numpy as jnp
from jax.experimental import pallas as pl
from jax.experimental.pallas import tpu as pltpu

B, K, V = 32, 8, 100000
PAD_TOKEN_ID = 0
KP = K + 1
LT = 128
FULL = V // LT
REM = V - FULL * LT
UNROLL = 8
NLOOP = (FULL - 1) // UNROLL
IBIG = 2**31 - 1


def _scan_kernel(tp_ref, ti_ref, did_ref, dp_ref, tid_out, tps_out):
    lane2d = jax.lax.broadcasted_iota(jnp.int32, (8, LT), 1)

    for q in range(B // 8):
        r0 = 8 * q
        did = did_ref[0, r0:r0 + 8, :1]
        dp = dp_ref[0, r0:r0 + 8, :1]

        def tile(t):
            tp = tp_ref[0, r0:r0 + 8, pl.ds(t * LT, LT)]
            ti = ti_ref[0, r0:r0 + 8, pl.ds(t * LT, LT)]
            eq = ti == did
            c = jnp.maximum(tp - jnp.where(eq, dp, 0.0), 0.0)
            msel = jnp.where(eq, tp, 0.0)
            return ti, c, msel

        def update(state, t, ti, c, msel):
            s_max, s_vt, s_ti, s_sum, s_tps = state
            s_sum = s_sum + c
            s_tps = s_tps + msel
            upd = c > s_max
            s_max = jnp.where(upd, c, s_max)
            s_vt = jnp.where(upd, t, s_vt)
            s_ti = jnp.where(upd, ti, s_ti)
            return (s_max, s_vt, s_ti, s_sum, s_tps)

        ti0, c0, msel0 = tile(0)
        state = (c0, jnp.zeros((8, LT), jnp.int32), ti0, c0, msel0)

        def body(i, state):
            for u in range(UNROLL):
                t = 1 + i * UNROLL + u
                state = update(state, t, *tile(t))
            return state

        state = jax.lax.fori_loop(0, NLOOP, body, state)
        for t in range(1 + NLOOP * UNROLL, FULL):
            state = update(state, t, *tile(t))

        tp_r = tp_ref[0, r0:r0 + 8, pl.ds(FULL * LT, REM)]
        ti_r = ti_ref[0, r0:r0 + 8, pl.ds(FULL * LT, REM)]
        eq_r = ti_r == did
        c_r = jnp.maximum(tp_r - jnp.where(eq_r, dp, 0.0), 0.0)
        msel_r = jnp.where(eq_r, tp_r, 0.0)
        zf = jnp.zeros((8, LT - REM), jnp.float32)
        zi = jnp.zeros((8, LT - REM), jnp.int32)
        c = jnp.concatenate([c_r, zf], axis=1)
        msel = jnp.concatenate([msel_r, zf], axis=1)
        ti = jnp.concatenate([ti_r, zi], axis=1)
        s_max, s_vt, s_ti, s_sum, s_tps = update(state, FULL, ti, c, msel)

        m = jnp.max(s_max, axis=1, keepdims=True)
        colg = s_vt * LT + lane2d
        idx = jnp.min(jnp.where(s_max == m, colg, IBIG), axis=1,
                      keepdims=True)
        tiv = jnp.sum(jnp.where(colg == idx, s_ti, 0), axis=1,
                      keepdims=True)
        ssum = jnp.sum(s_sum, axis=1, keepdims=True)
        tps = jnp.sum(s_tps, axis=1, keepdims=True)
        ti0c = jnp.sum(jnp.where(colg == 0, s_ti, 0), axis=1,
                       keepdims=True)
        tid = jnp.where(ssum < 1e-30, ti0c, tiv)
        tid_out[0, r0:r0 + 8, :] = jnp.broadcast_to(tid, (8, LT))
        tps_out[0, r0:r0 + 8, :] = jnp.broadcast_to(tps, (8, LT))


def _tail_kernel(tid_ref, tps_ref, did_ref, dp_ref, rnd_ref,
                 tok_out, idx_out):
    tid = tid_ref[...]
    tps = tps_ref[...]
    did = did_ref[...]
    dp = dp_ref[...]
    rnd = rnd_ref[...]

    ratio = jnp.minimum(tps / dp, 1.0)
    acc = (rnd < ratio).astype(jnp.float32)

    r_i = jax.lax.broadcasted_iota(jnp.int32, (16, 16), 0)
    c_i = jax.lax.broadcasted_iota(jnp.int32, (16, 16), 1)
    lt = (r_i <= c_i).astype(jnp.float32)
    cs = jnp.dot(acc, lt, preferred_element_type=jnp.float32)

    lane = jax.lax.broadcasted_iota(jnp.int32, (32, 16), 1)
    positions = (lane + 1).astype(jnp.float32)
    mask = cs == positions
    index = jnp.sum(mask.astype(jnp.int32), axis=1, keepdims=True)

    tokens = jnp.where(mask, did, tid)
    keep = index >= lane
    tokens = jnp.where(keep, tokens, PAD_TOKEN_ID)

    tok_out[...] = tokens
    idx_out[...] = jnp.broadcast_to(index, (32, 16))


@jax.jit
def kernel(draft_ids, draft_probs, target_probs, target_indices):
    tp_t = jnp.transpose(target_probs, (1, 0, 2))
    ti_t = jnp.transpose(target_indices, (1, 0, 2))

    didT = jnp.concatenate(
        [draft_ids, jnp.full((B, 1), -1, jnp.int32)], axis=1).T
    dpT = jnp.concatenate(
        [draft_probs, jnp.zeros((B, 1), jnp.float32)], axis=1).T
    did_b = jnp.broadcast_to(didT[:, :, None], (KP, B, 128))
    dp_b = jnp.broadcast_to(dpT[:, :, None], (KP, B, 128))

    tid_full, tps_full = pl.pallas_call(
        _scan_kernel,
        grid=(KP,),
        in_specs=[
            pl.BlockSpec((1, B, V), lambda k: (k, 0, 0)),
            pl.BlockSpec((1, B, V), lambda k: (k, 0, 0)),
            pl.BlockSpec((1, B, 128), lambda k: (k, 0, 0)),
            pl.BlockSpec((1, B, 128), lambda k: (k, 0, 0)),
        ],
        out_specs=[
            pl.BlockSpec((1, B, 128), lambda k: (k, 0, 0)),
            pl.BlockSpec((1, B, 128), lambda k: (k, 0, 0)),
        ],
        out_shape=[
            jax.ShapeDtypeStruct((KP, B, 128), jnp.int32),
            jax.ShapeDtypeStruct((KP, B, 128), jnp.float32),
        ],
        compiler_params=pltpu.CompilerParams(
            dimension_semantics=("arbitrary",)),
    )(tp_t, ti_t, did_b, dp_b)

    target_ids = tid_full[:, :, 0].T
    tp_sel = tps_full[:, :, 0].T[:, :K]

    tid_p = jnp.pad(target_ids, ((0, 0), (0, 16 - KP)))
    tps_p = jnp.pad(tp_sel, ((0, 0), (0, 16 - K)))
    did_p = jnp.pad(draft_ids, ((0, 0), (0, 16 - K)))
    dp_p = jnp.pad(draft_probs, ((0, 0), (0, 16 - K)), constant_values=1.0)
    rnd = jax.random.uniform(jax.random.key(42), (B, K), dtype=jnp.float32)
    rnd_p = jnp.pad(rnd, ((0, 0), (0, 16 - K)), constant_values=2.0)

    tokens16, idx16 = pl.pallas_call(
        _tail_kernel,
        out_shape=[
            jax.ShapeDtypeStruct((B, 16), jnp.int32),
            jax.ShapeDtypeStruct((B, 16), jnp.int32),
        ],
    )(tid_p, tps_p, did_p, dp_p, rnd_p)

    tokens = tokens16[:, :KP]
    index = idx16[:, :1]
    return (tokens, index)


# speedup vs baseline: 35.8372x; 1.1005x over previous
import jax
import jax.numpy as jnp
from jax.experimental import pallas as pl
from jax.experimental.pallas import tpu as pltpu

B, K, V = 32, 8, 100000
PAD_TOKEN_ID = 0
KP = K + 1
LT = 128
FULL = V // LT
REM = V - FULL * LT
UNROLL = 16
NLOOP = (FULL - 1) // UNROLL
IBIG = 2**31 - 1


def _scan_kernel(tp_ref, ti_ref, did_ref, dp_ref, tid_out, tps_out):
    lane2d = jax.lax.broadcasted_iota(jnp.int32, (8, LT), 1)

    for q in range(B // 8):
        r0 = 8 * q
        did = did_ref[0, r0:r0 + 8, :1]
        dp = dp_ref[0, r0:r0 + 8, :1]

        def tile(t):
            tp = tp_ref[0, r0:r0 + 8, pl.ds(t * LT, LT)]
            ti = ti_ref[0, r0:r0 + 8, pl.ds(t * LT, LT)]
            eq = ti == did
            c = jnp.maximum(tp - jnp.where(eq, dp, 0.0), 0.0)
            msel = jnp.where(eq, tp, 0.0)
            return ti, c, msel

        def update(state, t, ti, c, msel):
            s_max, s_vt, s_ti, s_sum, s_tps = state
            s_sum = s_sum + c
            s_tps = s_tps + msel
            upd = c > s_max
            s_max = jnp.where(upd, c, s_max)
            s_vt = jnp.where(upd, t, s_vt)
            s_ti = jnp.where(upd, ti, s_ti)
            return (s_max, s_vt, s_ti, s_sum, s_tps)

        ti0, c0, msel0 = tile(0)
        state = (c0, jnp.zeros((8, LT), jnp.int32), ti0, c0, msel0)

        def body(i, state):
            for u in range(UNROLL):
                t = 1 + i * UNROLL + u
                state = update(state, t, *tile(t))
            return state

        state = jax.lax.fori_loop(0, NLOOP, body, state)
        for t in range(1 + NLOOP * UNROLL, FULL):
            state = update(state, t, *tile(t))

        tp_r = tp_ref[0, r0:r0 + 8, pl.ds(FULL * LT, REM)]
        ti_r = ti_ref[0, r0:r0 + 8, pl.ds(FULL * LT, REM)]
        eq_r = ti_r == did
        c_r = jnp.maximum(tp_r - jnp.where(eq_r, dp, 0.0), 0.0)
        msel_r = jnp.where(eq_r, tp_r, 0.0)
        zf = jnp.zeros((8, LT - REM), jnp.float32)
        zi = jnp.zeros((8, LT - REM), jnp.int32)
        c = jnp.concatenate([c_r, zf], axis=1)
        msel = jnp.concatenate([msel_r, zf], axis=1)
        ti = jnp.concatenate([ti_r, zi], axis=1)
        s_max, s_vt, s_ti, s_sum, s_tps = update(state, FULL, ti, c, msel)

        m = jnp.max(s_max, axis=1, keepdims=True)
        colg = s_vt * LT + lane2d
        idx = jnp.min(jnp.where(s_max == m, colg, IBIG), axis=1,
                      keepdims=True)
        tiv = jnp.sum(jnp.where(colg == idx, s_ti, 0), axis=1,
                      keepdims=True)
        ssum = jnp.sum(s_sum, axis=1, keepdims=True)
        tps = jnp.sum(s_tps, axis=1, keepdims=True)
        ti0c = jnp.sum(jnp.where(colg == 0, s_ti, 0), axis=1,
                       keepdims=True)
        tid = jnp.where(ssum < 1e-30, ti0c, tiv)
        tid_out[0, r0:r0 + 8, :] = jnp.broadcast_to(tid, (8, LT))
        tps_out[0, r0:r0 + 8, :] = jnp.broadcast_to(tps, (8, LT))


def _tail_kernel(tid_ref, tps_ref, did_ref, dp_ref, rnd_ref,
                 tok_out, idx_out):
    tid = tid_ref[...]
    tps = tps_ref[...]
    did = did_ref[...]
    dp = dp_ref[...]
    rnd = rnd_ref[...]

    ratio = jnp.minimum(tps / dp, 1.0)
    acc = (rnd < ratio).astype(jnp.float32)

    r_i = jax.lax.broadcasted_iota(jnp.int32, (16, 16), 0)
    c_i = jax.lax.broadcasted_iota(jnp.int32, (16, 16), 1)
    lt = (r_i <= c_i).astype(jnp.float32)
    cs = jnp.dot(acc, lt, preferred_element_type=jnp.float32)

    lane = jax.lax.broadcasted_iota(jnp.int32, (32, 16), 1)
    positions = (lane + 1).astype(jnp.float32)
    mask = cs == positions
    index = jnp.sum(mask.astype(jnp.int32), axis=1, keepdims=True)

    tokens = jnp.where(mask, did, tid)
    keep = index >= lane
    tokens = jnp.where(keep, tokens, PAD_TOKEN_ID)

    tok_out[...] = tokens
    idx_out[...] = jnp.broadcast_to(index, (32, 16))


@jax.jit
def kernel(draft_ids, draft_probs, target_probs, target_indices):
    tp_t = jnp.transpose(target_probs, (1, 0, 2))
    ti_t = jnp.transpose(target_indices, (1, 0, 2))

    didT = jnp.concatenate(
        [draft_ids, jnp.full((B, 1), -1, jnp.int32)], axis=1).T
    dpT = jnp.concatenate(
        [draft_probs, jnp.zeros((B, 1), jnp.float32)], axis=1).T
    did_b = jnp.broadcast_to(didT[:, :, None], (KP, B, 128))
    dp_b = jnp.broadcast_to(dpT[:, :, None], (KP, B, 128))

    tid_full, tps_full = pl.pallas_call(
        _scan_kernel,
        grid=(KP,),
        in_specs=[
            pl.BlockSpec((1, B, V), lambda k: (k, 0, 0)),
            pl.BlockSpec((1, B, V), lambda k: (k, 0, 0)),
            pl.BlockSpec((1, B, 128), lambda k: (k, 0, 0)),
            pl.BlockSpec((1, B, 128), lambda k: (k, 0, 0)),
        ],
        out_specs=[
            pl.BlockSpec((1, B, 128), lambda k: (k, 0, 0)),
            pl.BlockSpec((1, B, 128), lambda k: (k, 0, 0)),
        ],
        out_shape=[
            jax.ShapeDtypeStruct((KP, B, 128), jnp.int32),
            jax.ShapeDtypeStruct((KP, B, 128), jnp.float32),
        ],
        compiler_params=pltpu.CompilerParams(
            dimension_semantics=("arbitrary",)),
    )(tp_t, ti_t, did_b, dp_b)

    target_ids = tid_full[:, :, 0].T
    tp_sel = tps_full[:, :, 0].T[:, :K]

    tid_p = jnp.pad(target_ids, ((0, 0), (0, 16 - KP)))
    tps_p = jnp.pad(tp_sel, ((0, 0), (0, 16 - K)))
    did_p = jnp.pad(draft_ids, ((0, 0), (0, 16 - K)))
    dp_p = jnp.pad(draft_probs, ((0, 0), (0, 16 - K)), constant_values=1.0)
    rnd = jax.random.uniform(jax.random.key(42), (B, K), dtype=jnp.float32)
    rnd_p = jnp.pad(rnd, ((0, 0), (0, 16 - K)), constant_values=2.0)

    tokens16, idx16 = pl.pallas_call(
        _tail_kernel,
        out_shape=[
            jax.ShapeDtypeStruct((B, 16), jnp.int32),
            jax.ShapeDtypeStruct((B, 16), jnp.int32),
        ],
    )(tid_p, tps_p, did_p, dp_p, rnd_p)

    tokens = tokens16[:, :KP]
    index = idx16[:, :1]
    return (tokens, index)
